# baseline (speedup 1.0000x reference)
import functools

import jax
import jax.numpy as jnp
from jax import lax
from jax.experimental import pallas as pl
from jax.experimental.pallas import tpu as pltpu
from jax.experimental.pallas import tpu_sc as plsc

N = 10000
E = 320000
D_IN = 128
H = 64
C = 32

NW = 32
K = 128
TOT_CH = 2560
E_PAD = TOT_CH * K
NP = 10240
RPT = NP // 16
PAD_ROWS = NP - N
R = 2048
NBUF = 4
DEPTH = 2

NCH_A = 80
NCH_B = 80
REPL = 4
CH_A_TOT = 16 * NCH_A
NCH_MAX = max(NCH_A, NCH_B)
NCH_DEG = TOT_CH // NW


def _mesh():
    return plsc.VectorSubcoreMesh(core_axis_name="c", subcore_axis_name="s")


@functools.partial(
    pl.kernel,
    mesh=_mesh(),
    out_type=jax.ShapeDtypeStruct((NW, NP // 16, 16), jnp.float32),
    scratch_types=[
        pltpu.VMEM((NCH_DEG, K), jnp.int32),
        pltpu.VMEM((NP // 16, 16), jnp.float32),
    ],
    compiler_params=pltpu.CompilerParams(needs_layout_passes=False),
)
def _deg_kernel(dst_hbm, degp_hbm, dstv, degv):
    c = lax.axis_index("c")
    s = lax.axis_index("s")
    wid = s * 2 + c
    zeros = jnp.zeros((16,), jnp.float32)

    def zbody(i, carry):
        degv[i, :] = zeros
        return carry

    lax.fori_loop(0, NP // 16, zbody, 0)
    pltpu.sync_copy(dst_hbm.at[pl.ds(wid * NCH_DEG, NCH_DEG)], dstv)
    ones = jnp.ones((16,), jnp.float32)

    def cbody(j, carry):
        for k in range(K // 16):
            idx = dstv[j, pl.ds(k * 16, 16)]
            plsc.addupdate_scatter(degv, [idx >> 4, idx & 15], ones)
        return carry

    lax.fori_loop(0, NCH_DEG, cbody, 0)
    pltpu.sync_copy(degv, degp_hbm.at[wid])


@functools.partial(
    pl.kernel,
    mesh=_mesh(),
    out_type=jax.ShapeDtypeStruct((2, NP, H), jnp.float32),
    scratch_types=[
        pltpu.VMEM((NCH_MAX, K), jnp.int32),
        pltpu.VMEM((NCH_MAX, K), jnp.int32),
        pltpu.VMEM((NBUF, K, H), jnp.float32),
        pltpu.VMEM_SHARED((NP, H), jnp.float32),
        [pltpu.SemaphoreType.DMA] * NBUF,
        [pltpu.SemaphoreType.DMA] * NBUF,
    ],
    compiler_params=pltpu.CompilerParams(
        needs_layout_passes=False, use_tc_tiling_on_sc=False
    ),
)
def _agg_kernel(g_hbm, src_hbm, dst_hbm, part_hbm, srcv, dstv, rows, acc, gsem, ssem):
    c = lax.axis_index("c")
    s = lax.axis_index("s")
    pltpu.sync_copy(g_hbm.at[pl.ds(s * RPT, RPT)], acc.at[pl.ds(s * RPT, RPT)])
    plsc.subcore_barrier()

    def _pipeline(nch, base):
        pltpu.async_copy(
            src_hbm.at[pl.ds(base, nch)], srcv.at[pl.ds(0, nch)], gsem[0]
        )
        pltpu.async_copy(
            dst_hbm.at[pl.ds(base, nch)], dstv.at[pl.ds(0, nch)], ssem[0]
        )
        pltpu.make_async_copy(
            src_hbm.at[pl.ds(base, nch)], srcv.at[pl.ds(0, nch)], gsem[0]
        ).wait()
        pltpu.make_async_copy(
            dst_hbm.at[pl.ds(base, nch)], dstv.at[pl.ds(0, nch)], ssem[0]
        ).wait()
        for j in range(DEPTH):
            pltpu.async_copy(g_hbm.at[srcv.at[j]], rows.at[j], gsem[j])

        def body(i4, carry):
            for u in range(NBUF):
                j = i4 * NBUF + u
                b = u
                bg = (u + DEPTH) % NBUF
                jg = j + DEPTH

                @pl.when(j >= NBUF - DEPTH)
                def _free():
                    pltpu.make_async_copy(
                        rows.at[bg], acc.at[dstv.at[j]], ssem[bg]
                    ).wait()

                @pl.when(jg < nch)
                def _prefetch():
                    pltpu.async_copy(g_hbm.at[srcv.at[jg]], rows.at[bg], gsem[bg])

                pltpu.make_async_copy(
                    g_hbm.at[srcv.at[j]], rows.at[b], gsem[b]
                ).wait()
                pltpu.async_copy(rows.at[b], acc.at[dstv.at[j]], ssem[b], add=True)
            return carry

        lax.fori_loop(0, nch // NBUF, body, 0)
        for j in range(nch - DEPTH, nch):
            b = j % NBUF
            pltpu.make_async_copy(rows.at[b], acc.at[dstv.at[j]], ssem[b]).wait()

    @pl.when(c == 0)
    def _core_a():
        _pipeline(NCH_A, s * NCH_A)

    @pl.when(c == 1)
    def _core_b():
        _pipeline(NCH_B, CH_A_TOT + s * NCH_B)

    plsc.subcore_barrier()
    pltpu.sync_copy(acc.at[pl.ds(s * RPT, RPT)], part_hbm.at[c].at[pl.ds(s * RPT, RPT)])


def _tc_prologue(x_p, degp, wtn, bn, wt1):
    def body(x_b, degp_b, wtn_b, bn_b, wt1_b, g1_b, dinv_b):
        deg = jnp.sum(degp_b[...], axis=0)[:, None] + 1.0
        dinv = lax.rsqrt(deg)
        h0 = jnp.dot(x_b[...], wtn_b[...], preferred_element_type=jnp.float32) + bn_b[...]
        g1 = dinv * jnp.dot(h0, wt1_b[...], preferred_element_type=jnp.float32)
        g1_b[...] = jnp.broadcast_to(g1[None], (REPL, R, H))
        dinv_b[...] = jnp.broadcast_to(dinv, (R, H))

    return pl.pallas_call(
        body,
        grid=(NP // R,),
        in_specs=[
            pl.BlockSpec((R, D_IN), lambda i: (i, 0)),
            pl.BlockSpec((NW, R), lambda i: (0, i)),
            pl.BlockSpec((D_IN, H), lambda i: (0, 0)),
            pl.BlockSpec((1, H), lambda i: (0, 0)),
            pl.BlockSpec((H, H), lambda i: (0, 0)),
        ],
        out_specs=[
            pl.BlockSpec((REPL, R, H), lambda i: (0, i, 0)),
            pl.BlockSpec((R, H), lambda i: (i, 0)),
        ],
        out_shape=[
            jax.ShapeDtypeStruct((REPL, NP, H), jnp.float32),
            jax.ShapeDtypeStruct((NP, H), jnp.float32),
        ],
    )(x_p, degp, wtn, bn, wt1)


def _tc_combine(parts, g, dinv64, b, wt):
    def body(p_b, g_b, d_b, b_b, wt_b, o_b):
        sagg = p_b[0] + p_b[1] - g_b[0]
        h = jnp.maximum(d_b[...] * sagg + b_b[...], 0.0)
        gn = d_b[...] * jnp.dot(h, wt_b[...], preferred_element_type=jnp.float32)
        o_b[...] = jnp.broadcast_to(gn[None], (REPL, R, H))

    return pl.pallas_call(
        body,
        grid=(NP // R,),
        in_specs=[
            pl.BlockSpec((2, R, H), lambda i: (0, i, 0)),
            pl.BlockSpec((1, R, H), lambda i: (0, i, 0)),
            pl.BlockSpec((R, H), lambda i: (i, 0)),
            pl.BlockSpec((1, H), lambda i: (0, 0)),
            pl.BlockSpec((H, H), lambda i: (0, 0)),
        ],
        out_specs=pl.BlockSpec((REPL, R, H), lambda i: (0, i, 0)),
        out_shape=jax.ShapeDtypeStruct((REPL, NP, H), jnp.float32),
    )(parts, g, dinv64, b, wt)


def _tc_epilogue(parts, g, dinv64, b3, wtp, bp):
    def body(p_b, g_b, d_b, b_b, wtp_b, bp_b, o_b):
        sagg = p_b[0] + p_b[1] - g_b[0]
        h = jnp.maximum(d_b[...] * sagg + b_b[...], 0.0)
        o_b[...] = jnp.dot(h, wtp_b[...], preferred_element_type=jnp.float32) + bp_b[...]

    return pl.pallas_call(
        body,
        grid=(NP // R,),
        in_specs=[
            pl.BlockSpec((2, R, H), lambda i: (0, i, 0)),
            pl.BlockSpec((1, R, H), lambda i: (0, i, 0)),
            pl.BlockSpec((R, H), lambda i: (i, 0)),
            pl.BlockSpec((1, H), lambda i: (0, 0)),
            pl.BlockSpec((H, C), lambda i: (0, 0)),
            pl.BlockSpec((1, C), lambda i: (0, 0)),
        ],
        out_specs=pl.BlockSpec((R, C), lambda i: (i, 0)),
        out_shape=jax.ShapeDtypeStruct((NP, C), jnp.float32),
    )(parts, g, dinv64, b3, wtp, bp)


def kernel(x, edge_index, edge_attr, batch, W_node, b_node, W1, b1, W2, b2, W3, b3, W_post, b_post):
    del edge_attr, batch
    src = edge_index[0]
    dst = edge_index[1]
    pad_e = E_PAD - E
    pad_src = jnp.zeros((pad_e,), jnp.int32)
    pad_dst = N + (jnp.arange(pad_e, dtype=jnp.int32) % PAD_ROWS)
    src_p = jnp.concatenate([src, pad_src]).reshape(TOT_CH, K)
    dst_p = jnp.concatenate([dst, pad_dst]).reshape(TOT_CH, K)
    rep = (jnp.arange(TOT_CH, dtype=jnp.int32) // NCH_DEG) % REPL
    src_p = src_p + rep[:, None] * NP
    x_p = jnp.pad(x, ((0, NP - N), (0, 0)))

    degp = _deg_kernel(dst_p).reshape(NW, NP)
    g1, dinv64 = _tc_prologue(x_p, degp, W_node.T, b_node[None], W1.T)
    parts1 = _agg_kernel(g1.reshape(REPL * NP, H), src_p, dst_p)
    g2 = _tc_combine(parts1, g1, dinv64, b1[None], W2.T)
    parts2 = _agg_kernel(g2.reshape(REPL * NP, H), src_p, dst_p)
    g3 = _tc_combine(parts2, g2, dinv64, b2[None], W3.T)
    parts3 = _agg_kernel(g3.reshape(REPL * NP, H), src_p, dst_p)
    out = _tc_epilogue(parts3, g3, dinv64, b3[None], W_post.T, b_post[None])
    return out[:N]


# speedup vs baseline: 2.6512x; 2.6512x over previous
import functools

import jax
import jax.numpy as jnp
from jax import lax
from jax.experimental import pallas as pl
from jax.experimental.pallas import tpu as pltpu
from jax.experimental.pallas import tpu_sc as plsc

N = 10000
E = 320000
D_IN = 128
H = 64
HH = H // 2
C = 32

NW = 32
K = 128
TOT_CH = 2560
E_PAD = TOT_CH * K
NP = 10240
RPT = NP // 16
PAD_ROWS = NP - N
R = 2048
NBUF = 4
DEPTH = 2
NCH = TOT_CH // NW


def _mesh():
    return plsc.VectorSubcoreMesh(core_axis_name="c", subcore_axis_name="s")


@functools.partial(
    pl.kernel,
    mesh=_mesh(),
    out_type=jax.ShapeDtypeStruct((NW, NP // 16, 16), jnp.float32),
    scratch_types=[
        pltpu.VMEM((NCH, K), jnp.int32),
        pltpu.VMEM((NP // 16, 16), jnp.float32),
    ],
    compiler_params=pltpu.CompilerParams(needs_layout_passes=False),
)
def _deg_kernel(dst_hbm, degp_hbm, dstv, degv):
    c = lax.axis_index("c")
    s = lax.axis_index("s")
    wid = s * 2 + c
    zeros = jnp.zeros((16,), jnp.float32)

    def zbody(i, carry):
        degv[i, :] = zeros
        return carry

    lax.fori_loop(0, NP // 16, zbody, 0)
    pltpu.sync_copy(dst_hbm.at[pl.ds(wid * NCH, NCH)], dstv)
    ones = jnp.ones((16,), jnp.float32)

    def cbody(j, carry):
        for k in range(K // 16):
            idx = dstv[j, pl.ds(k * 16, 16)]
            plsc.addupdate_scatter(degv, [idx >> 4, idx & 15], ones)
        return carry

    lax.fori_loop(0, NCH, cbody, 0)
    pltpu.sync_copy(degv, degp_hbm.at[wid])


@functools.partial(
    pl.kernel,
    mesh=_mesh(),
    out_type=jax.ShapeDtypeStruct((4, NP, HH), jnp.float32),
    scratch_types=[
        pltpu.VMEM((NCH, K), jnp.int32),
        pltpu.VMEM((NCH, K), jnp.int32),
        pltpu.VMEM((NBUF, K, HH), jnp.float32),
        pltpu.VMEM_SHARED((NP, HH), jnp.float32),
        pltpu.VMEM_SHARED((NP, HH), jnp.float32),
        [pltpu.SemaphoreType.DMA] * NBUF,
        [pltpu.SemaphoreType.DMA] * NBUF,
    ],
    compiler_params=pltpu.CompilerParams(
        needs_layout_passes=False, use_tc_tiling_on_sc=False
    ),
)
def _agg_kernel(ga_hbm, gb_hbm, src_hbm, dst_hbm, part_hbm, srcv, dstv, rows, acc, g_sp, gsem, ssem):
    c = lax.axis_index("c")
    s = lax.axis_index("s")
    wid = s * 2 + c
    base = wid * NCH
    pltpu.async_copy(src_hbm.at[pl.ds(base, NCH)], srcv, gsem[0])
    pltpu.async_copy(dst_hbm.at[pl.ds(base, NCH)], dstv, ssem[0])
    pltpu.make_async_copy(src_hbm.at[pl.ds(base, NCH)], srcv, gsem[0]).wait()
    pltpu.make_async_copy(dst_hbm.at[pl.ds(base, NCH)], dstv, ssem[0]).wait()

    for half in range(2):
        gh_hbm = (ga_hbm, gb_hbm)[half]
        sl = pl.ds(s * RPT, RPT)
        pltpu.sync_copy(gh_hbm.at[sl], g_sp.at[sl])
        pltpu.sync_copy(gh_hbm.at[sl], acc.at[sl])
        plsc.subcore_barrier()

        for j in range(DEPTH):
            pltpu.async_copy(g_sp.at[srcv.at[j]], rows.at[j], gsem[j])

        def body(i4, carry):
            for u in range(NBUF):
                j = i4 * NBUF + u
                b = u
                bg = (u + DEPTH) % NBUF
                jg = j + DEPTH

                @pl.when(j >= NBUF - DEPTH)
                def _free():
                    pltpu.make_async_copy(
                        rows.at[bg], acc.at[dstv.at[j]], ssem[bg]
                    ).wait()

                @pl.when(jg < NCH)
                def _prefetch():
                    pltpu.async_copy(g_sp.at[srcv.at[jg]], rows.at[bg], gsem[bg])

                pltpu.make_async_copy(
                    g_sp.at[srcv.at[j]], rows.at[b], gsem[b]
                ).wait()
                pltpu.async_copy(rows.at[b], acc.at[dstv.at[j]], ssem[b], add=True)
            return carry

        lax.fori_loop(0, NCH // NBUF, body, 0)
        for j in range(NCH - DEPTH, NCH):
            b = j % NBUF
            pltpu.make_async_copy(rows.at[b], acc.at[dstv.at[j]], ssem[b]).wait()
        plsc.subcore_barrier()
        pltpu.sync_copy(acc.at[sl], part_hbm.at[c * 2 + half].at[sl])


def _tc_prologue(x_p, degp, wtn, bn, wt1):
    def body(x_b, degp_b, wtn_b, bn_b, wt1_b, ga_b, gb_b, dinv_b):
        deg = jnp.sum(degp_b[...], axis=0)[:, None] + 1.0
        dinv = lax.rsqrt(deg)
        h0 = jnp.dot(x_b[...], wtn_b[...], preferred_element_type=jnp.float32) + bn_b[...]
        g1 = dinv * jnp.dot(h0, wt1_b[...], preferred_element_type=jnp.float32)
        ga_b[...] = g1[:, :HH]
        gb_b[...] = g1[:, HH:]
        dinv_b[...] = jnp.broadcast_to(dinv, (R, H))

    return pl.pallas_call(
        body,
        grid=(NP // R,),
        in_specs=[
            pl.BlockSpec((R, D_IN), lambda i: (i, 0)),
            pl.BlockSpec((NW, R), lambda i: (0, i)),
            pl.BlockSpec((D_IN, H), lambda i: (0, 0)),
            pl.BlockSpec((1, H), lambda i: (0, 0)),
            pl.BlockSpec((H, H), lambda i: (0, 0)),
        ],
        out_specs=[
            pl.BlockSpec((R, HH), lambda i: (i, 0)),
            pl.BlockSpec((R, HH), lambda i: (i, 0)),
            pl.BlockSpec((R, H), lambda i: (i, 0)),
        ],
        out_shape=[
            jax.ShapeDtypeStruct((NP, HH), jnp.float32),
            jax.ShapeDtypeStruct((NP, HH), jnp.float32),
            jax.ShapeDtypeStruct((NP, H), jnp.float32),
        ],
    )(x_p, degp, wtn, bn, wt1)


def _combine_body(p_b, ga_b, gb_b):
    sa = p_b[0] + p_b[2] - ga_b[...]
    sb = p_b[1] + p_b[3] - gb_b[...]
    return jnp.concatenate([sa, sb], axis=-1)


def _tc_combine(parts, ga, gb, dinv64, b, wt):
    def body(p_b, ga_b, gb_b, d_b, b_b, wt_b, oa_b, ob_b):
        sagg = _combine_body(p_b, ga_b, gb_b)
        h = jnp.maximum(d_b[...] * sagg + b_b[...], 0.0)
        gn = d_b[...] * jnp.dot(h, wt_b[...], preferred_element_type=jnp.float32)
        oa_b[...] = gn[:, :HH]
        ob_b[...] = gn[:, HH:]

    return pl.pallas_call(
        body,
        grid=(NP // R,),
        in_specs=[
            pl.BlockSpec((4, R, HH), lambda i: (0, i, 0)),
            pl.BlockSpec((R, HH), lambda i: (i, 0)),
            pl.BlockSpec((R, HH), lambda i: (i, 0)),
            pl.BlockSpec((R, H), lambda i: (i, 0)),
            pl.BlockSpec((1, H), lambda i: (0, 0)),
            pl.BlockSpec((H, H), lambda i: (0, 0)),
        ],
        out_specs=[
            pl.BlockSpec((R, HH), lambda i: (i, 0)),
            pl.BlockSpec((R, HH), lambda i: (i, 0)),
        ],
        out_shape=[
            jax.ShapeDtypeStruct((NP, HH), jnp.float32),
            jax.ShapeDtypeStruct((NP, HH), jnp.float32),
        ],
    )(parts, ga, gb, dinv64, b, wt)


def _tc_epilogue(parts, ga, gb, dinv64, b3, wtp, bp):
    def body(p_b, ga_b, gb_b, d_b, b_b, wtp_b, bp_b, o_b):
        sagg = _combine_body(p_b, ga_b, gb_b)
        h = jnp.maximum(d_b[...] * sagg + b_b[...], 0.0)
        o_b[...] = jnp.dot(h, wtp_b[...], preferred_element_type=jnp.float32) + bp_b[...]

    return pl.pallas_call(
        body,
        grid=(NP // R,),
        in_specs=[
            pl.BlockSpec((4, R, HH), lambda i: (0, i, 0)),
            pl.BlockSpec((R, HH), lambda i: (i, 0)),
            pl.BlockSpec((R, HH), lambda i: (i, 0)),
            pl.BlockSpec((R, H), lambda i: (i, 0)),
            pl.BlockSpec((1, H), lambda i: (0, 0)),
            pl.BlockSpec((H, C), lambda i: (0, 0)),
            pl.BlockSpec((1, C), lambda i: (0, 0)),
        ],
        out_specs=pl.BlockSpec((R, C), lambda i: (i, 0)),
        out_shape=jax.ShapeDtypeStruct((NP, C), jnp.float32),
    )(parts, ga, gb, dinv64, b3, wtp, bp)


def kernel(x, edge_index, edge_attr, batch, W_node, b_node, W1, b1, W2, b2, W3, b3, W_post, b_post):
    del edge_attr, batch
    src = edge_index[0]
    dst = edge_index[1]
    pad_e = E_PAD - E
    pad_src = jnp.arange(pad_e, dtype=jnp.int32) % N
    pad_dst = N + (jnp.arange(pad_e, dtype=jnp.int32) % PAD_ROWS)
    src_p = jnp.concatenate([src, pad_src]).reshape(TOT_CH, K)
    dst_p = jnp.concatenate([dst, pad_dst]).reshape(TOT_CH, K)
    x_p = jnp.pad(x, ((0, NP - N), (0, 0)))

    degp = _deg_kernel(dst_p).reshape(NW, NP)
    g1a, g1b, dinv64 = _tc_prologue(x_p, degp, W_node.T, b_node[None], W1.T)
    parts1 = _agg_kernel(g1a, g1b, src_p, dst_p)
    g2a, g2b = _tc_combine(parts1, g1a, g1b, dinv64, b1[None], W2.T)
    parts2 = _agg_kernel(g2a, g2b, src_p, dst_p)
    g3a, g3b = _tc_combine(parts2, g2a, g2b, dinv64, b2[None], W3.T)
    parts3 = _agg_kernel(g3a, g3b, src_p, dst_p)
    out = _tc_epilogue(parts3, g3a, g3b, dinv64, b3[None], W_post.T, b_post[None])
    return out[:N]


# speedup vs baseline: 3.3436x; 1.2611x over previous
import functools

import jax
import jax.numpy as jnp
from jax import lax
from jax.experimental import pallas as pl
from jax.experimental.pallas import tpu as pltpu
from jax.experimental.pallas import tpu_sc as plsc

N = 10000
E = 320000
D_IN = 128
H = 64
HH = H // 2
C = 32

NW = 32
K = 128
TOT_CH = 2560
E_PAD = TOT_CH * K
NP = 10240
RPT = NP // 16
PAD_ROWS = NP - N
R = 2048
NBUF = 4
DEPTH = 2
NCH = TOT_CH // NW


def _mesh():
    return plsc.VectorSubcoreMesh(core_axis_name="c", subcore_axis_name="s")


@functools.partial(
    pl.kernel,
    mesh=_mesh(),
    out_type=jax.ShapeDtypeStruct((NW, NP // 16, 16), jnp.float32),
    scratch_types=[
        pltpu.VMEM((NCH, K), jnp.int32),
        pltpu.VMEM((NP // 16, 16), jnp.float32),
    ],
    compiler_params=pltpu.CompilerParams(needs_layout_passes=False),
)
def _deg_kernel(dst_hbm, degp_hbm, dstv, degv):
    c = lax.axis_index("c")
    s = lax.axis_index("s")
    wid = s * 2 + c
    zeros = jnp.zeros((16,), jnp.float32)

    def zbody(i, carry):
        degv[i, :] = zeros
        return carry

    lax.fori_loop(0, NP // 16, zbody, 0)
    pltpu.sync_copy(dst_hbm.at[pl.ds(wid * NCH, NCH)], dstv)
    ones = jnp.ones((16,), jnp.float32)

    def cbody(j, carry):
        for k in range(K // 16):
            idx = dstv[j, pl.ds(k * 16, 16)]
            plsc.addupdate_scatter(degv, [idx >> 4, idx & 15], ones)
        return carry

    lax.fori_loop(0, NCH, cbody, 0)
    pltpu.sync_copy(degv, degp_hbm.at[wid])


@functools.partial(
    pl.kernel,
    mesh=_mesh(),
    out_type=jax.ShapeDtypeStruct((4, NP, HH), jnp.float32),
    scratch_types=[
        pltpu.VMEM((NCH, K), jnp.int32),
        pltpu.VMEM((NCH, K), jnp.int32),
        pltpu.VMEM((NBUF, K, HH), jnp.float32),
        pltpu.VMEM_SHARED((NP, HH), jnp.float32),
        pltpu.VMEM_SHARED((NP, HH), jnp.float32),
        [pltpu.SemaphoreType.DMA] * NBUF,
        [pltpu.SemaphoreType.DMA] * NBUF,
    ],
    compiler_params=pltpu.CompilerParams(
        needs_layout_passes=False, use_tc_tiling_on_sc=False
    ),
)
def _agg_kernel(ga_hbm, gb_hbm, src_hbm, dst_hbm, part_hbm, srcv, dstv, rows, acc, g_sp, gsem, ssem):
    c = lax.axis_index("c")
    s = lax.axis_index("s")
    wid = s * 2 + c
    base = wid * NCH
    pltpu.async_copy(src_hbm.at[pl.ds(base, NCH)], srcv, gsem[0])
    pltpu.async_copy(dst_hbm.at[pl.ds(base, NCH)], dstv, ssem[0])
    pltpu.make_async_copy(src_hbm.at[pl.ds(base, NCH)], srcv, gsem[0]).wait()
    pltpu.make_async_copy(dst_hbm.at[pl.ds(base, NCH)], dstv, ssem[0]).wait()

    for half in range(2):
        gh_hbm = (ga_hbm, gb_hbm)[half]
        sl = pl.ds(s * RPT, RPT)
        pltpu.sync_copy(gh_hbm.at[sl], g_sp.at[sl])
        pltpu.sync_copy(gh_hbm.at[sl], acc.at[sl])
        plsc.subcore_barrier()

        for j in range(DEPTH):
            pltpu.async_copy(g_sp.at[srcv.at[j]], rows.at[j], gsem[j])

        def body(i4, carry):
            for u in range(NBUF):
                j = i4 * NBUF + u
                b = u
                bg = (u + DEPTH) % NBUF
                jg = j + DEPTH

                @pl.when(j >= NBUF - DEPTH)
                def _free():
                    pltpu.make_async_copy(
                        rows.at[bg], acc.at[dstv.at[j]], ssem[bg]
                    ).wait()

                @pl.when(jg < NCH)
                def _prefetch():
                    pltpu.async_copy(g_sp.at[srcv.at[jg]], rows.at[bg], gsem[bg])

                pltpu.make_async_copy(
                    g_sp.at[srcv.at[j]], rows.at[b], gsem[b]
                ).wait()
                pltpu.async_copy(rows.at[b], acc.at[dstv.at[j]], ssem[b], add=True)
            return carry

        lax.fori_loop(0, NCH // NBUF, body, 0)
        for j in range(NCH - DEPTH, NCH):
            b = j % NBUF
            pltpu.make_async_copy(rows.at[b], acc.at[dstv.at[j]], ssem[b]).wait()
        plsc.subcore_barrier()
        pltpu.sync_copy(acc.at[sl], part_hbm.at[c * 2 + half].at[sl])


def _tc_prologue(x_p, degp, wtn, bn, wt1):
    def body(x_b, degp_b, wtn_b, bn_b, wt1_b, ga_b, gb_b, dinv_b):
        deg = jnp.sum(degp_b[...], axis=0)[:, None] + 1.0
        dinv = lax.rsqrt(deg)
        h0 = jnp.dot(x_b[...], wtn_b[...], preferred_element_type=jnp.float32) + bn_b[...]
        g1 = dinv * jnp.dot(h0, wt1_b[...], preferred_element_type=jnp.float32)
        ga_b[...] = g1[:, :HH]
        gb_b[...] = g1[:, HH:]
        dinv_b[...] = jnp.broadcast_to(dinv, (R, HH))

    return pl.pallas_call(
        body,
        grid=(NP // R,),
        in_specs=[
            pl.BlockSpec((R, D_IN), lambda i: (i, 0)),
            pl.BlockSpec((NW, R), lambda i: (0, i)),
            pl.BlockSpec((D_IN, H), lambda i: (0, 0)),
            pl.BlockSpec((1, H), lambda i: (0, 0)),
            pl.BlockSpec((H, H), lambda i: (0, 0)),
        ],
        out_specs=[
            pl.BlockSpec((R, HH), lambda i: (i, 0)),
            pl.BlockSpec((R, HH), lambda i: (i, 0)),
            pl.BlockSpec((R, HH), lambda i: (i, 0)),
        ],
        out_shape=[
            jax.ShapeDtypeStruct((NP, HH), jnp.float32),
            jax.ShapeDtypeStruct((NP, HH), jnp.float32),
            jax.ShapeDtypeStruct((NP, HH), jnp.float32),
        ],
    )(x_p, degp, wtn, bn, wt1)


PKN = NP * HH // 128
PKR = R * HH // 128


def _tc_combine(parts, ga, gb, dinv_pk, ba, bb, bd_aa, bd_ab, bd_ba, bd_bb):
    def body(p_b, ga_b, gb_b, d_b, ba_b, bb_b, aa_b, ab_b, ba2_b, bb2_b, oa_b, ob_b):
        d = d_b[...]
        sa = p_b[0] + p_b[2] - ga_b[...]
        sb = p_b[1] + p_b[3] - gb_b[...]
        hda = d * jnp.maximum(d * sa + ba_b[...], 0.0)
        hdb = d * jnp.maximum(d * sb + bb_b[...], 0.0)
        dot = lambda a, w: jnp.dot(a, w, preferred_element_type=jnp.float32)
        oa_b[...] = dot(hda, aa_b[...]) + dot(hdb, ba2_b[...])
        ob_b[...] = dot(hda, ab_b[...]) + dot(hdb, bb2_b[...])

    return pl.pallas_call(
        body,
        grid=(NP // R,),
        in_specs=[
            pl.BlockSpec((4, PKR, 128), lambda i: (0, i, 0)),
            pl.BlockSpec((PKR, 128), lambda i: (i, 0)),
            pl.BlockSpec((PKR, 128), lambda i: (i, 0)),
            pl.BlockSpec((PKR, 128), lambda i: (i, 0)),
            pl.BlockSpec((1, 128), lambda i: (0, 0)),
            pl.BlockSpec((1, 128), lambda i: (0, 0)),
            pl.BlockSpec((128, 128), lambda i: (0, 0)),
            pl.BlockSpec((128, 128), lambda i: (0, 0)),
            pl.BlockSpec((128, 128), lambda i: (0, 0)),
            pl.BlockSpec((128, 128), lambda i: (0, 0)),
        ],
        out_specs=[
            pl.BlockSpec((PKR, 128), lambda i: (i, 0)),
            pl.BlockSpec((PKR, 128), lambda i: (i, 0)),
        ],
        out_shape=[
            jax.ShapeDtypeStruct((PKN, 128), jnp.float32),
            jax.ShapeDtypeStruct((PKN, 128), jnp.float32),
        ],
    )(parts, ga, gb, dinv_pk, ba, bb, bd_aa, bd_ab, bd_ba, bd_bb)


def _tc_epilogue(parts, ga, gb, dinv_pk, ba, bb, bdp_a, bdp_b, bp):
    def body(p_b, ga_b, gb_b, d_b, ba_b, bb_b, pa_b, pb_b, bp_b, o_b):
        d = d_b[...]
        sa = p_b[0] + p_b[2] - ga_b[...]
        sb = p_b[1] + p_b[3] - gb_b[...]
        ha = jnp.maximum(d * sa + ba_b[...], 0.0)
        hb = jnp.maximum(d * sb + bb_b[...], 0.0)
        dot = lambda a, w: jnp.dot(a, w, preferred_element_type=jnp.float32)
        o_b[...] = dot(ha, pa_b[...]) + dot(hb, pb_b[...]) + bp_b[...]

    return pl.pallas_call(
        body,
        grid=(NP // R,),
        in_specs=[
            pl.BlockSpec((4, PKR, 128), lambda i: (0, i, 0)),
            pl.BlockSpec((PKR, 128), lambda i: (i, 0)),
            pl.BlockSpec((PKR, 128), lambda i: (i, 0)),
            pl.BlockSpec((PKR, 128), lambda i: (i, 0)),
            pl.BlockSpec((1, 128), lambda i: (0, 0)),
            pl.BlockSpec((1, 128), lambda i: (0, 0)),
            pl.BlockSpec((128, 128), lambda i: (0, 0)),
            pl.BlockSpec((128, 128), lambda i: (0, 0)),
            pl.BlockSpec((1, 128), lambda i: (0, 0)),
        ],
        out_specs=pl.BlockSpec((PKR, 128), lambda i: (i, 0)),
        out_shape=jax.ShapeDtypeStruct((PKN, 128), jnp.float32),
    )(parts, ga, gb, dinv_pk, ba, bb, bdp_a, bdp_b, bp)


def kernel(x, edge_index, edge_attr, batch, W_node, b_node, W1, b1, W2, b2, W3, b3, W_post, b_post):
    del edge_attr, batch
    src = edge_index[0]
    dst = edge_index[1]
    pad_e = E_PAD - E
    pad_src = jnp.arange(pad_e, dtype=jnp.int32) % N
    pad_dst = N + (jnp.arange(pad_e, dtype=jnp.int32) % PAD_ROWS)
    src_p = jnp.concatenate([src, pad_src]).reshape(TOT_CH, K)
    dst_p = jnp.concatenate([dst, pad_dst]).reshape(TOT_CH, K)
    x_p = jnp.pad(x, ((0, NP - N), (0, 0)))

    pk = (4, PKN, 128)
    eye4 = jnp.eye(4, dtype=jnp.float32)
    bd = lambda w: jnp.kron(eye4, w)
    pkb = lambda v: jnp.tile(v, 4)[None]
    wt2, wt3, wtp = W2.T, W3.T, W_post.T

    degp = _deg_kernel(dst_p).reshape(NW, NP)
    g1a, g1b, dinv32 = _tc_prologue(x_p, degp, W_node.T, b_node[None], W1.T)
    dinv_pk = dinv32.reshape(PKN, 128)
    parts1 = _agg_kernel(g1a, g1b, src_p, dst_p).reshape(pk)
    g2a, g2b = _tc_combine(
        parts1, g1a.reshape(PKN, 128), g1b.reshape(PKN, 128), dinv_pk,
        pkb(b1[:HH]), pkb(b1[HH:]),
        bd(wt2[:HH, :HH]), bd(wt2[:HH, HH:]), bd(wt2[HH:, :HH]), bd(wt2[HH:, HH:]))
    parts2 = _agg_kernel(g2a.reshape(NP, HH), g2b.reshape(NP, HH), src_p, dst_p).reshape(pk)
    g3a, g3b = _tc_combine(
        parts2, g2a, g2b, dinv_pk,
        pkb(b2[:HH]), pkb(b2[HH:]),
        bd(wt3[:HH, :HH]), bd(wt3[:HH, HH:]), bd(wt3[HH:, :HH]), bd(wt3[HH:, HH:]))
    parts3 = _agg_kernel(g3a.reshape(NP, HH), g3b.reshape(NP, HH), src_p, dst_p).reshape(pk)
    out = _tc_epilogue(
        parts3, g3a, g3b, dinv_pk,
        pkb(b3[:HH]), pkb(b3[HH:]),
        bd(wtp[:HH, :]), bd(wtp[HH:, :]), pkb(b_post))
    return out.reshape(NP, C)[:N]
